# baseline (speedup 1.0000x reference)
import functools

import jax
import jax.numpy as jnp
from jax import lax
from jax.experimental import pallas as pl
from jax.experimental.pallas import tpu as pltpu
from jax.experimental.pallas import tpu_sc as plsc

BATCH = 16384
D = 16
L = 16
RPL = 8
CHUNK = 128
NLINE = (1000000 * D) // 128


@functools.lru_cache(maxsize=1)
def _build():
    info = plsc.get_sparse_core_info()
    nc, ns = info.num_cores, info.num_subcores
    nw = nc * ns
    bpw = BATCH // nw
    nchunk = bpw // CHUNK
    mesh = plsc.VectorSubcoreMesh(core_axis_name="c", subcore_axis_name="s")

    @functools.partial(
        pl.kernel,
        mesh=mesh,
        compiler_params=pltpu.CompilerParams(needs_layout_passes=False),
        out_type=jax.ShapeDtypeStruct((BATCH,), jnp.float32),
        scratch_types=[
            pltpu.VMEM((nchunk, CHUNK), jnp.int32),
            pltpu.VMEM((nchunk, CHUNK), jnp.int32),
            pltpu.VMEM((nchunk, CHUNK), jnp.int32),
            pltpu.VMEM((nchunk, CHUNK), jnp.int32),
            pltpu.VMEM((2, CHUNK, 128), jnp.float32),
            pltpu.VMEM((2, CHUNK, 128), jnp.float32),
            pltpu.VMEM((bpw,), jnp.float32),
            pltpu.SemaphoreType.DMA,
            pltpu.SemaphoreType.DMA,
        ],
    )
    def k(idx_i_hbm, idx_j_hbm, table_hbm, out_hbm,
          idxi_v, idxj_v, lni_v, lnj_v, lines_i, lines_j, out_v,
          sem_i, sem_j):
        wid = lax.axis_index("s") * nc + lax.axis_index("c")
        base = wid * bpw
        for c in range(nchunk):
            pltpu.sync_copy(idx_i_hbm.at[pl.ds(base + c * CHUNK, CHUNK)],
                            idxi_v.at[c])
            pltpu.sync_copy(idx_j_hbm.at[pl.ds(base + c * CHUNK, CHUNK)],
                            idxj_v.at[c])
        for c in range(nchunk):
            for g in range(CHUNK // L):
                s = pl.ds(g * L, L)
                lni_v.at[c][s] = lax.shift_right_logical(idxi_v.at[c][s], 3)
                lnj_v.at[c][s] = lax.shift_right_logical(idxj_v.at[c][s], 3)

        def fire(c, slot):
            return (pltpu.async_copy(table_hbm.at[lni_v.at[c]],
                                     lines_i.at[slot], sem_i),
                    pltpu.async_copy(table_hbm.at[lnj_v.at[c]],
                                     lines_j.at[slot], sem_j))

        inflight = fire(0, 0)
        for c in range(nchunk):
            nxt = fire(c + 1, (c + 1) % 2) if c + 1 < nchunk else None
            inflight[0].wait()
            inflight[1].wait()
            slot = c % 2
            for g in range(CHUNK // L):
                s = pl.ds(g * L, L)
                bvec = g * L + lax.iota(jnp.int32, L)
                ri = (idxi_v.at[c][s] & 7) * D
                rj = (idxj_v.at[c][s] & 7) * D
                acc = jnp.zeros((L,), jnp.float32)
                for kk in range(D):
                    a = plsc.load_gather(lines_i.at[slot], [bvec, ri + kk])
                    b = plsc.load_gather(lines_j.at[slot], [bvec, rj + kk])
                    acc = acc + a * b
                out_v[pl.ds(c * CHUNK + g * L, L)] = acc
            inflight = nxt
        pltpu.sync_copy(out_v, out_hbm.at[pl.ds(base, bpw)])

    return k


def kernel(user_index_i, user_index_j, user_embedding):
    k = _build()
    table_lines = user_embedding.reshape(NLINE, 128)
    return k(user_index_i.astype(jnp.int32),
             user_index_j.astype(jnp.int32),
             table_lines)


# speedup vs baseline: 2.1351x; 2.1351x over previous
import functools

import jax
import jax.numpy as jnp
from jax import lax
from jax.experimental import pallas as pl
from jax.experimental.pallas import tpu as pltpu
from jax.experimental.pallas import tpu_sc as plsc

BATCH = 16384
D = 16
L = 16
RPT = 8
NTILE = 1000000 // RPT
G = 2


@functools.lru_cache(maxsize=1)
def _build():
    info = plsc.get_sparse_core_info()
    nc, ns = info.num_cores, info.num_subcores
    nw = nc * ns
    bpw = BATCH // nw
    nbody = bpw // (G * L)
    mesh = plsc.VectorSubcoreMesh(core_axis_name="c", subcore_axis_name="s")

    @functools.partial(
        pl.kernel,
        mesh=mesh,
        compiler_params=pltpu.CompilerParams(
            needs_layout_passes=False, use_tc_tiling_on_sc=True),
        out_type=jax.ShapeDtypeStruct((BATCH,), jnp.float32),
        scratch_types=[
            pltpu.VMEM((bpw,), jnp.int32),
            pltpu.VMEM((bpw,), jnp.int32),
            pltpu.VMEM((G * L, RPT, D), jnp.float32),
            pltpu.VMEM((G * L, RPT, D), jnp.float32),
            pltpu.VMEM((bpw,), jnp.float32),
            pltpu.SemaphoreType.DMA,
            pltpu.SemaphoreType.DMA,
        ],
    )
    def k(idx_i_hbm, idx_j_hbm, table_hbm, out_hbm,
          idxi_v, idxj_v, tiles_i, tiles_j, out_v, sem_i, sem_j):
        wid = lax.axis_index("s") * nc + lax.axis_index("c")
        base = wid * bpw
        pltpu.sync_copy(idx_i_hbm.at[pl.ds(base, bpw)], idxi_v)
        pltpu.sync_copy(idx_j_hbm.at[pl.ds(base, bpw)], idxj_v)

        @pl.loop(0, nbody)
        def body(it):
            b0 = it * (G * L)
            copies = []
            for g in range(G):
                s = pl.ds(b0 + g * L, L)
                tiv = lax.shift_right_logical(idxi_v[s], 3)
                tjv = lax.shift_right_logical(idxj_v[s], 3)
                for t in range(L):
                    copies.append(pltpu.async_copy(
                        table_hbm.at[tiv[t]], tiles_i.at[g * L + t], sem_i))
                    copies.append(pltpu.async_copy(
                        table_hbm.at[tjv[t]], tiles_j.at[g * L + t], sem_j))
            for cp in copies:
                cp.wait()
            for g in range(G):
                s = pl.ds(b0 + g * L, L)
                bvec = g * L + lax.iota(jnp.int32, L)
                ri = idxi_v[s] & 7
                rj = idxj_v[s] & 7
                acc = jnp.zeros((L,), jnp.float32)
                for kk in range(D):
                    col = jnp.full((L,), kk, jnp.int32)
                    a = plsc.load_gather(tiles_i, [bvec, ri, col])
                    b = plsc.load_gather(tiles_j, [bvec, rj, col])
                    acc = acc + a * b
                out_v[s] = acc

        pltpu.sync_copy(out_v, out_hbm.at[pl.ds(base, bpw)])

    return k


def kernel(user_index_i, user_index_j, user_embedding):
    k = _build()
    table_tiles = user_embedding.reshape(NTILE, RPT, D)
    return k(user_index_i.astype(jnp.int32),
             user_index_j.astype(jnp.int32),
             table_tiles)
